# baseline (speedup 1.0000x reference)
import functools

import jax
import jax.numpy as jnp
from jax import lax
from jax.experimental import pallas as pl
from jax.experimental.pallas import tpu as pltpu
from jax.experimental.pallas import tpu_sc as plsc

N_NODES = 10000
F = 128
NC = 2
NS = 16
NW = NC * NS
CHUNK = 128
BK = 8
DUMMY_ROWS = 16
DEG_W = 8


def _sc_mesh():
    return plsc.VectorSubcoreMesh(
        core_axis_name="c", subcore_axis_name="s", num_cores=NC, num_subcores=NS
    )


def _make_deg_kernel(n_nodes, nblk):
    mesh = _sc_mesh()

    @functools.partial(
        pl.kernel,
        out_type=jax.ShapeDtypeStruct((NC, n_nodes, F), jnp.float32),
        mesh=mesh,
        scratch_types=[
            pltpu.VMEM((nblk, BK, CHUNK), jnp.int32),
            pltpu.VMEM((CHUNK, F), jnp.float32),
            pltpu.VMEM_SHARED((n_nodes + DUMMY_ROWS, F), jnp.float32),
            pltpu.SemaphoreType.DMA,
        ],
    )
    def deg_kernel(dst_hbm, ones_hbm, zeros_hbm, out_hbm, dst_v, ones_v, table,
                   sem):
        cid = lax.axis_index("c")
        sid = lax.axis_index("s")
        wid = cid * NS + sid

        rpt = (n_nodes // NS) // 8 * 8
        rem = n_nodes - rpt * NS
        sl = pl.ds(sid * rpt, rpt)
        pltpu.sync_copy(zeros_hbm.at[sl], table.at[sl])
        if rem:
            @pl.when(sid == NS - 1)
            def _():
                rsl = pl.ds(NS * rpt, rem)
                pltpu.sync_copy(zeros_hbm.at[rsl], table.at[rsl])
        pltpu.sync_copy(ones_hbm, ones_v)
        pltpu.sync_copy(dst_hbm.at[wid], dst_v)
        plsc.subcore_barrier()

        def body(k, carry):
            for j in range(BK):
                pltpu.async_copy(ones_v, table.at[dst_v.at[k, j]], sem,
                                 add=True)

            @pl.when(k >= 1)
            def _():
                for j in range(BK):
                    pltpu.make_async_copy(ones_v, table.at[dst_v.at[0, 0]],
                                          sem).wait()
            return carry

        lax.fori_loop(0, nblk, body, 0)
        for _ in range(BK):
            pltpu.make_async_copy(ones_v, table.at[dst_v.at[0, 0]], sem).wait()
        plsc.subcore_barrier()

        @pl.when(sid == 0)
        def _():
            pltpu.sync_copy(table.at[pl.ds(0, n_nodes)], out_hbm.at[cid])

    return deg_kernel


def _make_agg_kernel(n_nodes, nblk):
    mesh = _sc_mesh()

    @functools.partial(
        pl.kernel,
        out_type=jax.ShapeDtypeStruct((NC, n_nodes, F), jnp.float32),
        mesh=mesh,
        scratch_types=[
            pltpu.VMEM((2, BK, CHUNK), jnp.int32),
            pltpu.VMEM((2, BK, CHUNK), jnp.int32),
            pltpu.VMEM((CHUNK, F), jnp.float32),
            pltpu.VMEM((CHUNK, F), jnp.float32),
            pltpu.VMEM_SHARED((n_nodes + DUMMY_ROWS, F), jnp.float32),
            pltpu.SemaphoreType.DMA,
            pltpu.SemaphoreType.DMA,
            pltpu.SemaphoreType.DMA,
            pltpu.SemaphoreType.DMA,
            pltpu.SemaphoreType.DMA,
        ],
    )
    def agg_kernel(hp_hbm, src_hbm, dst_hbm, zeros_hbm, out_hbm,
                   srg, dsg, rows0, rows1, acc, si, sg0, sg1, ss0, ss1):
        cid = lax.axis_index("c")
        sid = lax.axis_index("s")
        wid = cid * NS + sid

        rpt = (n_nodes // NS) // 8 * 8
        rem = n_nodes - rpt * NS
        sl = pl.ds(sid * rpt, rpt)
        rsl = pl.ds(NS * rpt, rem)

        @pl.when(cid == 0)
        def _():
            pltpu.sync_copy(hp_hbm.at[sl], acc.at[sl])

        @pl.when(cid != 0)
        def _():
            pltpu.sync_copy(zeros_hbm.at[sl], acc.at[sl])

        if rem:
            @pl.when(sid == NS - 1)
            def _():
                @pl.when(cid == 0)
                def _():
                    pltpu.sync_copy(hp_hbm.at[rsl], acc.at[rsl])

                @pl.when(cid != 0)
                def _():
                    pltpu.sync_copy(zeros_hbm.at[rsl], acc.at[rsl])

        pltpu.async_copy(src_hbm.at[wid, 0], srg.at[0], si)
        pltpu.async_copy(dst_hbm.at[wid, 0], dsg.at[0], si)
        plsc.subcore_barrier()
        pltpu.make_async_copy(src_hbm.at[wid, 0], srg.at[0], si).wait()
        pltpu.make_async_copy(dst_hbm.at[wid, 0], dsg.at[0], si).wait()

        bufs = ((rows0, sg0, ss0), (rows1, sg1, ss1))
        pltpu.async_copy(hp_hbm.at[srg.at[0, 0]], rows0, sg0)
        pltpu.async_copy(hp_hbm.at[srg.at[0, 1]], rows1, sg1)

        assert BK >= 4 and BK % 2 == 0

        def blk(k, carry):
            slot = lax.rem(k, 2)
            nslot = lax.rem(k + 1, 2)
            for j in range(BK):
                rows_b, sg, ss = bufs[j % 2]
                if j == 2:
                    @pl.when(k + 1 < nblk)
                    def _():
                        kn = jnp.minimum(k + 1, nblk - 1)
                        pltpu.async_copy(src_hbm.at[wid, kn], srg.at[nslot],
                                         si)
                        pltpu.async_copy(dst_hbm.at[wid, kn], dsg.at[nslot],
                                         si)
                if j == BK - 2:
                    @pl.when(k + 1 < nblk)
                    def _():
                        pltpu.make_async_copy(src_hbm.at[wid, 0], srg.at[0],
                                              si).wait()
                        pltpu.make_async_copy(dst_hbm.at[wid, 0], dsg.at[0],
                                              si).wait()
                pltpu.make_async_copy(hp_hbm.at[srg.at[slot, j]], rows_b,
                                      sg).wait()
                pltpu.async_copy(rows_b, acc.at[dsg.at[slot, j]], ss,
                                 add=True)
                if j + 2 < BK:
                    pltpu.make_async_copy(rows_b, acc.at[dsg.at[slot, j]],
                                          ss).wait()
                    pltpu.async_copy(hp_hbm.at[srg.at[slot, j + 2]], rows_b,
                                     sg)
                else:
                    jj = j + 2 - BK
                    @pl.when(k + 1 < nblk)
                    def _():
                        pltpu.make_async_copy(rows_b, acc.at[dsg.at[slot, j]],
                                              ss).wait()
                        pltpu.async_copy(hp_hbm.at[srg.at[nslot, jj]], rows_b,
                                         sg)
            return carry

        lax.fori_loop(0, nblk, blk, 0)
        for rows_b, sg, ss in bufs:
            pltpu.make_async_copy(rows_b, acc.at[dsg.at[0, 0]], ss).wait()
        plsc.subcore_barrier()

        pltpu.sync_copy(acc.at[sl], out_hbm.at[cid, sl])
        if rem:
            @pl.when(sid == NS - 1)
            def _():
                pltpu.sync_copy(acc.at[rsl], out_hbm.at[cid, rsl])

    return agg_kernel


BLK = 1000


def _dinv(d0_ref, d1_ref):
    deg = d0_ref[:, 0:1] + d1_ref[:, 0:1] + 1.0
    return lax.rsqrt(deg)


def _stage_a_body(d0_ref, d1_ref, x_ref, w_ref, hp_ref):
    dinv = _dinv(d0_ref, d1_ref)
    hp_ref[...] = jnp.dot(
        x_ref[...] * dinv, w_ref[...], preferred_element_type=jnp.float32
    )


def _stage_b_body(d0_ref, d1_ref, a0_ref, a1_ref, b_ref, w_ref, hp_ref):
    dinv = _dinv(d0_ref, d1_ref)
    s = a0_ref[...] + a1_ref[...]
    t = jnp.maximum(dinv * s + b_ref[...], 0.0)
    hp_ref[...] = jnp.dot(
        t * dinv, w_ref[...], preferred_element_type=jnp.float32
    )


def _stage_c_body(d0_ref, d1_ref, a0_ref, a1_ref, b_ref, w_ref, bc_ref, o_ref):
    dinv = _dinv(d0_ref, d1_ref)
    s = a0_ref[...] + a1_ref[...]
    t = jnp.maximum(dinv * s + b_ref[...], 0.0)
    o_ref[...] = jnp.dot(
        t, w_ref[...], preferred_element_type=jnp.float32
    ) + bc_ref[...]


def _row_spec(width):
    return pl.BlockSpec((BLK, width), lambda i: (i, 0))


def _full_spec(shape):
    return pl.BlockSpec(shape, lambda i: tuple(0 for _ in shape))


def _stage_a(d0, d1, x, w):
    n = x.shape[0]
    return pl.pallas_call(
        _stage_a_body,
        grid=(n // BLK,),
        in_specs=[_row_spec(DEG_W), _row_spec(DEG_W), _row_spec(F),
                  _full_spec((F, F))],
        out_specs=_row_spec(F),
        out_shape=jax.ShapeDtypeStruct((n, F), jnp.float32),
    )(d0, d1, x, w)


def _stage_b(d0, d1, a0, a1, b, w):
    n = a0.shape[0]
    return pl.pallas_call(
        _stage_b_body,
        grid=(n // BLK,),
        in_specs=[_row_spec(DEG_W), _row_spec(DEG_W), _row_spec(F),
                  _row_spec(F), _full_spec((1, F)), _full_spec((F, F))],
        out_specs=_row_spec(F),
        out_shape=jax.ShapeDtypeStruct((n, F), jnp.float32),
    )(d0, d1, a0, a1, b, w)


def _stage_c(d0, d1, a0, a1, b, w, bc):
    n = a0.shape[0]
    k = w.shape[1]
    return pl.pallas_call(
        _stage_c_body,
        grid=(n // BLK,),
        in_specs=[_row_spec(DEG_W), _row_spec(DEG_W), _row_spec(F),
                  _row_spec(F), _full_spec((1, F)), _full_spec((F, k)),
                  _full_spec((1, k))],
        out_specs=pl.BlockSpec((BLK, k), lambda i: (i, 0)),
        out_shape=jax.ShapeDtypeStruct((n, k), jnp.float32),
    )(d0, d1, a0, a1, b, w, bc)


def kernel(x, edge_index, W1, b1, W2, b2, Wc, bc):
    n = x.shape[0]
    src = edge_index[0].astype(jnp.int32)
    dst = edge_index[1].astype(jnp.int32)
    e = src.shape[0]

    per = NW * BK * CHUNK
    nblk = -(-e // per)
    e_pad = nblk * per
    if e_pad != e:
        src = jnp.concatenate(
            [src, jnp.zeros((e_pad - e,), jnp.int32)])
        dst = jnp.concatenate(
            [dst, jnp.full((e_pad - e,), n, jnp.int32)])
    src_r = src.reshape(NW, nblk, BK, CHUNK)
    dst_r = dst.reshape(NW, nblk, BK, CHUNK)

    zeros2 = jnp.zeros((n, F), jnp.float32)
    ones_deg = jnp.ones((CHUNK, F), jnp.float32)

    deg_kernel = _make_deg_kernel(n, nblk)
    agg_kernel = _make_agg_kernel(n, nblk)

    degp = deg_kernel(dst_r, ones_deg, zeros2)
    d0, d1 = degp[0, :, :DEG_W], degp[1, :, :DEG_W]

    hp1 = _stage_a(d0, d1, x, W1)
    acc1 = agg_kernel(hp1, src_r, dst_r, zeros2)
    hp2 = _stage_b(d0, d1, acc1[0], acc1[1], b1.reshape(1, F), W2)
    acc2 = agg_kernel(hp2, src_r, dst_r, zeros2)
    out = _stage_c(d0, d1, acc2[0], acc2[1], b2.reshape(1, F), Wc,
                   bc.reshape(1, -1))
    return out
